# baseline (speedup 1.0000x reference)
import functools

import jax
import jax.numpy as jnp
from jax.experimental import pallas as pl

Q, D, CAP, A, K_NN, H = 1024, 128, 100000, 8, 32, 64
CB = 2048
CAP_PAD = 100352
PAD_VAL = 1e17


def _dist_mlp_kernel(obs_ref, keysT_ref, w1t_ref, b1_ref, w2t_ref, b2_ref,
                     w3t_ref, b3_ref, dist_ref, qnet_ref):
    j = pl.program_id(0)

    kt = keysT_ref[...]
    ksq = jnp.sum(kt * kt, axis=0, keepdims=True)
    prod = jnp.dot(obs_ref[...], kt, preferred_element_type=jnp.float32)
    dist_ref[...] = ksq - 2.0 * prod

    @pl.when(j == 0)
    def _():
        h1 = jnp.maximum(
            jnp.dot(obs_ref[...], w1t_ref[...],
                    preferred_element_type=jnp.float32) + b1_ref[...], 0.0)
        h2 = jnp.maximum(
            jnp.dot(h1, w2t_ref[...],
                    preferred_element_type=jnp.float32) + b2_ref[...], 0.0)
        qnet_ref[...] = (jnp.dot(h2, w3t_ref[...],
                                 preferred_element_type=jnp.float32)
                         + b3_ref[...])


@jax.jit
def kernel(observation, keys, values, W1, b1, W2, b2, W3, b3):
    keys_p = jnp.pad(keys, ((0, CAP_PAD - CAP), (0, 0)),
                     constant_values=PAD_VAL)
    keysT = keys_p.T
    grid = CAP_PAD // CB

    dist, qnet = pl.pallas_call(
        _dist_mlp_kernel,
        grid=(grid,),
        in_specs=[
            pl.BlockSpec((Q, D), lambda j: (0, 0)),
            pl.BlockSpec((D, CB), lambda j: (0, j)),
            pl.BlockSpec((D, H), lambda j: (0, 0)),
            pl.BlockSpec((1, H), lambda j: (0, 0)),
            pl.BlockSpec((H, H), lambda j: (0, 0)),
            pl.BlockSpec((1, H), lambda j: (0, 0)),
            pl.BlockSpec((H, A), lambda j: (0, 0)),
            pl.BlockSpec((1, A), lambda j: (0, 0)),
        ],
        out_specs=[
            pl.BlockSpec((Q, CB), lambda j: (0, j)),
            pl.BlockSpec((Q, A), lambda j: (0, 0)),
        ],
        out_shape=[
            jax.ShapeDtypeStruct((Q, CAP_PAD), jnp.float32),
            jax.ShapeDtypeStruct((Q, A), jnp.float32),
        ],
    )(observation, keysT, W1.T, b1[None, :], W2.T, b2[None, :],
      W3.T, b3[None, :])

    _, idx = jax.lax.top_k(-dist, K_NN)
    qec = jnp.mean(jnp.take(values, idx, axis=0), axis=1)
    return jnp.argmax(qec + qnet, axis=-1)


# speedup vs baseline: 9.1058x; 9.1058x over previous
import functools

import jax
import jax.numpy as jnp
from jax import lax
from jax.experimental import pallas as pl
from jax.experimental.pallas import tpu as pltpu
from jax.experimental.pallas import tpu_sc as plsc

Q, D, CAP, A, K_NN, H = 1024, 128, 100000, 8, 32, 64
CB = 2048
CAP_PAD = 100352
NBLK = CAP_PAD // CB
GW = 128
G = CAP_PAD // GW
GPB = CB // GW
PAD_VAL = 1e17
INF = float(3e38)
BIGI = int(2**30)


def _stage_a(keys_ref, keysT_ref, obs_ref, obsT_ref, dist_ref, mt_ref):
    kb = keys_ref[...]
    ktb = keysT_ref[...]
    ksq_row = jnp.sum(ktb * ktb, axis=0, keepdims=True)
    dist_ref[...] = ksq_row - 2.0 * jnp.dot(
        obs_ref[...], ktb, preferred_element_type=jnp.float32)
    dist_t = ksq_row.T - 2.0 * jnp.dot(
        kb, obsT_ref[...], preferred_element_type=jnp.float32)
    mt_ref[...] = jnp.min(dist_t.reshape(GPB, GW, Q), axis=1)


def _stage_b(mt_ref, ids_ref, scr_ref):
    scr_ref[...] = mt_ref[...]
    iota_g = lax.broadcasted_iota(jnp.int32, (G, Q), 0)
    iota_q = lax.broadcasted_iota(jnp.int32, (1, Q), 1)
    for i in range(K_NN):
        x = scr_ref[...]
        mn = jnp.min(x, axis=0, keepdims=True)
        eq = x == mn
        gid = jnp.min(jnp.where(eq, iota_g, BIGI), axis=0, keepdims=True)
        scr_ref[...] = jnp.where(eq & (iota_g == gid), INF, x)
        ids_ref[pl.ds(i, 1), :] = iota_q * G + gid


def _sc_gather(dist_seg, ids_flat):
    info = plsc.get_sparse_core_info()
    nw = info.num_cores * info.num_subcores
    b = Q * K_NN
    b_per_w = b // nw
    chunk = 512
    mesh = plsc.VectorSubcoreMesh(core_axis_name="c", subcore_axis_name="s")

    @functools.partial(
        pl.kernel, mesh=mesh,
        out_type=jax.ShapeDtypeStruct((b, GW), jnp.float32),
        scratch_types=[
            pltpu.VMEM((chunk,), jnp.int32),
            pltpu.VMEM((chunk, GW), jnp.float32),
            pltpu.SemaphoreType.DMA,
        ],
    )
    def k(table_hbm, idx_hbm, out_hbm, idx_v, rows_v, sem):
        wid = lax.axis_index("s") * info.num_cores + lax.axis_index("c")
        for c in range(b_per_w // chunk):
            base = wid * b_per_w + c * chunk
            pltpu.sync_copy(idx_hbm.at[pl.ds(base, chunk)], idx_v)
            pltpu.async_copy(table_hbm.at[idx_v], rows_v, sem).wait()
            pltpu.sync_copy(rows_v, out_hbm.at[pl.ds(base, chunk)])

    return k(dist_seg, ids_flat)


def _stage_d(gath_ref, t_ref, scr_ref):
    scr_ref[...] = gath_ref[...]
    mn = jnp.zeros((Q, 1), jnp.float32)
    for _ in range(K_NN):
        x = scr_ref[...]
        mn = jnp.min(x, axis=1, keepdims=True)
        scr_ref[...] = jnp.where(x == mn, INF, x)
    t_ref[...] = jnp.broadcast_to(mn, (Q, 128))


def _stage_e(dist_ref, obs_ref, vals_ref, tst_ref,
             w1_ref, b1_ref, w2_ref, b2_ref, w3_ref, b3_ref,
             act_ref, s_ref):
    j = pl.program_id(0)
    mask = (dist_ref[...] <= tst_ref[:, 0:1]).astype(jnp.float32)
    contrib = jnp.dot(mask, vals_ref[...],
                      preferred_element_type=jnp.float32)

    @pl.when(j == 0)
    def _():
        s_ref[...] = jnp.zeros_like(s_ref)

    s_ref[...] += contrib

    @pl.when(j == NBLK - 1)
    def _():
        h1 = jnp.maximum(jnp.dot(obs_ref[...], w1_ref[...],
                                 preferred_element_type=jnp.float32)
                         + b1_ref[...], 0.0)
        h2 = jnp.maximum(jnp.dot(h1, w2_ref[...],
                                 preferred_element_type=jnp.float32)
                         + b2_ref[...], 0.0)
        qn = jnp.dot(h2, w3_ref[...],
                     preferred_element_type=jnp.float32) + b3_ref[...]
        qt = s_ref[:, 0:A] / jnp.float32(K_NN) + qn
        best = qt[:, 0:1]
        am = jnp.zeros((Q, 1), jnp.int32)
        for a in range(1, A):
            cond = qt[:, a:a + 1] > best
            am = jnp.where(cond, jnp.int32(a), am)
            best = jnp.where(cond, qt[:, a:a + 1], best)
        act_ref[...] = jnp.broadcast_to(am, (Q, 128))


@jax.jit
def kernel(observation, keys, values, W1, b1, W2, b2, W3, b3):
    keys_p = jnp.pad(keys, ((0, CAP_PAD - CAP), (0, 0)),
                     constant_values=PAD_VAL)
    keysT = keys_p.T
    obsT = observation.T
    vals_aug = jnp.concatenate(
        [values, jnp.ones((CAP, 1), jnp.float32)], axis=1)
    valsq = jnp.pad(vals_aug, ((0, CAP_PAD - CAP), (0, 16 - (A + 1))))

    dist, mt = pl.pallas_call(
        _stage_a,
        grid=(NBLK,),
        in_specs=[
            pl.BlockSpec((CB, D), lambda j: (j, 0)),
            pl.BlockSpec((D, CB), lambda j: (0, j)),
            pl.BlockSpec((Q, D), lambda j: (0, 0)),
            pl.BlockSpec((D, Q), lambda j: (0, 0)),
        ],
        out_specs=[
            pl.BlockSpec((Q, CB), lambda j: (0, j)),
            pl.BlockSpec((GPB, Q), lambda j: (j, 0)),
        ],
        out_shape=[
            jax.ShapeDtypeStruct((Q, CAP_PAD), jnp.float32),
            jax.ShapeDtypeStruct((G, Q), jnp.float32),
        ],
    )(keys_p, keysT, observation, obsT)

    ids_t = pl.pallas_call(
        _stage_b,
        scratch_shapes=[pltpu.VMEM((G, Q), jnp.float32)],
        out_shape=jax.ShapeDtypeStruct((K_NN, Q), jnp.int32),
    )(mt)

    ids_flat = ids_t.T.reshape(Q * K_NN)
    gath = _sc_gather(dist.reshape(Q * G, GW), ids_flat)

    tstar = pl.pallas_call(
        _stage_d,
        scratch_shapes=[pltpu.VMEM((Q, K_NN * GW), jnp.float32)],
        out_shape=jax.ShapeDtypeStruct((Q, 128), jnp.float32),
    )(gath.reshape(Q, K_NN * GW))

    act = pl.pallas_call(
        _stage_e,
        grid=(NBLK,),
        in_specs=[
            pl.BlockSpec((Q, CB), lambda j: (0, j)),
            pl.BlockSpec((Q, D), lambda j: (0, 0)),
            pl.BlockSpec((CB, 16), lambda j: (j, 0)),
            pl.BlockSpec((Q, 128), lambda j: (0, 0)),
            pl.BlockSpec((D, H), lambda j: (0, 0)),
            pl.BlockSpec((1, H), lambda j: (0, 0)),
            pl.BlockSpec((H, H), lambda j: (0, 0)),
            pl.BlockSpec((1, H), lambda j: (0, 0)),
            pl.BlockSpec((H, A), lambda j: (0, 0)),
            pl.BlockSpec((1, A), lambda j: (0, 0)),
        ],
        out_specs=pl.BlockSpec((Q, 128), lambda j: (0, 0)),
        out_shape=jax.ShapeDtypeStruct((Q, 128), jnp.int32),
        scratch_shapes=[pltpu.VMEM((Q, 16), jnp.float32)],
    )(dist, observation, valsq, tstar, W1.T, b1[None, :], W2.T, b2[None, :],
      W3.T, b3[None, :])

    return act[:, 0]


# speedup vs baseline: 12.5350x; 1.3766x over previous
import functools

import jax
import jax.numpy as jnp
from jax import lax
from jax.experimental import pallas as pl
from jax.experimental.pallas import tpu as pltpu
from jax.experimental.pallas import tpu_sc as plsc

Q, D, CAP, A, K_NN, H = 1024, 128, 100000, 8, 32, 64
CB = 2048
CAP_PAD = 100352
NBLK = CAP_PAD // CB
GW = 128
G = CAP_PAD // GW
GPB = CB // GW
PAD_VAL = 1e17
INF = float(3e38)
BIGI = int(2**30)


def _stage_a(keys_ref, keysT_ref, obs_ref, obsT_ref, dist_ref, mt_ref):
    kb = keys_ref[...]
    ktb = keysT_ref[...]
    ksq_row = jnp.sum(ktb * ktb, axis=0, keepdims=True)
    dist_q = ksq_row - 2.0 * jnp.dot(
        obs_ref[...], ktb, preferred_element_type=jnp.float32)
    dist_ref[...] = dist_q.reshape(Q, GPB, GW)
    dist_t = ksq_row.T - 2.0 * jnp.dot(
        kb, obsT_ref[...], preferred_element_type=jnp.float32)
    mt_ref[...] = jnp.min(dist_t.reshape(GPB, GW, Q), axis=1)


def _stage_b(mt_ref, ids_ref, scr_ref):
    scr_ref[...] = mt_ref[...]
    iota_g = lax.broadcasted_iota(jnp.int32, (G, Q), 0)
    iota_q = lax.broadcasted_iota(jnp.int32, (1, Q), 1)
    for i in range(K_NN):
        x = scr_ref[...]
        mn = jnp.min(x, axis=0, keepdims=True)
        eq = x == mn
        gid = jnp.min(jnp.where(eq, iota_g, BIGI), axis=0, keepdims=True)
        scr_ref[...] = jnp.where(eq & (iota_g == gid), INF, x)
        ids_ref[pl.ds(i, 1), :] = iota_q * G + gid


def _sc_gather(dist_seg, ids_flat):
    info = plsc.get_sparse_core_info()
    nw = info.num_cores * info.num_subcores
    b = Q * K_NN
    b_per_w = b // nw
    chunk = 512
    mesh = plsc.VectorSubcoreMesh(core_axis_name="c", subcore_axis_name="s")

    @functools.partial(
        pl.kernel, mesh=mesh,
        out_type=jax.ShapeDtypeStruct((b, GW), jnp.float32),
        scratch_types=[
            pltpu.VMEM((chunk,), jnp.int32),
            pltpu.VMEM((chunk, GW), jnp.float32),
            pltpu.SemaphoreType.DMA,
        ],
    )
    def k(table_hbm, idx_hbm, out_hbm, idx_v, rows_v, sem):
        wid = lax.axis_index("s") * info.num_cores + lax.axis_index("c")
        for c in range(b_per_w // chunk):
            base = wid * b_per_w + c * chunk
            pltpu.sync_copy(idx_hbm.at[pl.ds(base, chunk)], idx_v)
            pltpu.async_copy(table_hbm.at[idx_v], rows_v, sem).wait()
            pltpu.sync_copy(rows_v, out_hbm.at[pl.ds(base, chunk)])

    return k(dist_seg, ids_flat)


def _stage_d(gath_ref, t_ref, scr_ref):
    scr_ref[...] = gath_ref[...].reshape(Q, K_NN * GW)
    mn = jnp.zeros((Q, 1), jnp.float32)
    for _ in range(K_NN):
        x = scr_ref[...]
        mn = jnp.min(x, axis=1, keepdims=True)
        scr_ref[...] = jnp.where(x == mn, INF, x)
    t_ref[...] = jnp.broadcast_to(mn, (Q, 128))


def _stage_e(dist_ref, obs_ref, vals_ref, tst_ref,
             w1_ref, b1_ref, w2_ref, b2_ref, w3_ref, b3_ref,
             act_ref, s_ref):
    j = pl.program_id(0)
    dist_q = dist_ref[...].reshape(Q, CB)
    mask = (dist_q <= tst_ref[:, 0:1]).astype(jnp.float32)
    contrib = jnp.dot(mask, vals_ref[...],
                      preferred_element_type=jnp.float32)

    @pl.when(j == 0)
    def _():
        s_ref[...] = jnp.zeros_like(s_ref)

    s_ref[...] += contrib

    @pl.when(j == NBLK - 1)
    def _():
        h1 = jnp.maximum(jnp.dot(obs_ref[...], w1_ref[...],
                                 preferred_element_type=jnp.float32)
                         + b1_ref[...], 0.0)
        h2 = jnp.maximum(jnp.dot(h1, w2_ref[...],
                                 preferred_element_type=jnp.float32)
                         + b2_ref[...], 0.0)
        qn = jnp.dot(h2, w3_ref[...],
                     preferred_element_type=jnp.float32) + b3_ref[...]
        qt = s_ref[:, 0:A] / jnp.float32(K_NN) + qn
        best = qt[:, 0:1]
        am = jnp.zeros((Q, 1), jnp.int32)
        for a in range(1, A):
            cond = qt[:, a:a + 1] > best
            am = jnp.where(cond, jnp.int32(a), am)
            best = jnp.where(cond, qt[:, a:a + 1], best)
        act_ref[...] = jnp.broadcast_to(am, (Q, 128))


@jax.jit
def kernel(observation, keys, values, W1, b1, W2, b2, W3, b3):
    keys_p = jnp.pad(keys, ((0, CAP_PAD - CAP), (0, 0)),
                     constant_values=PAD_VAL)
    keysT = keys_p.T
    obsT = observation.T
    vals_aug = jnp.concatenate(
        [values, jnp.ones((CAP, 1), jnp.float32)], axis=1)
    valsq = jnp.pad(vals_aug, ((0, CAP_PAD - CAP), (0, 16 - (A + 1))))

    dist, mt = pl.pallas_call(
        _stage_a,
        grid=(NBLK,),
        in_specs=[
            pl.BlockSpec((CB, D), lambda j: (j, 0)),
            pl.BlockSpec((D, CB), lambda j: (0, j)),
            pl.BlockSpec((Q, D), lambda j: (0, 0)),
            pl.BlockSpec((D, Q), lambda j: (0, 0)),
        ],
        out_specs=[
            pl.BlockSpec((Q, GPB, GW), lambda j: (0, j, 0)),
            pl.BlockSpec((GPB, Q), lambda j: (j, 0)),
        ],
        out_shape=[
            jax.ShapeDtypeStruct((Q, G, GW), jnp.float32),
            jax.ShapeDtypeStruct((G, Q), jnp.float32),
        ],
    )(keys_p, keysT, observation, obsT)

    ids_t = pl.pallas_call(
        _stage_b,
        scratch_shapes=[pltpu.VMEM((G, Q), jnp.float32)],
        out_shape=jax.ShapeDtypeStruct((K_NN, Q), jnp.int32),
    )(mt)

    ids_flat = ids_t.T.reshape(Q * K_NN)
    gath = _sc_gather(dist.reshape(Q * G, GW), ids_flat)

    tstar = pl.pallas_call(
        _stage_d,
        scratch_shapes=[pltpu.VMEM((Q, K_NN * GW), jnp.float32)],
        out_shape=jax.ShapeDtypeStruct((Q, 128), jnp.float32),
    )(gath.reshape(Q, K_NN, GW))

    act = pl.pallas_call(
        _stage_e,
        grid=(NBLK,),
        in_specs=[
            pl.BlockSpec((Q, GPB, GW), lambda j: (0, j, 0)),
            pl.BlockSpec((Q, D), lambda j: (0, 0)),
            pl.BlockSpec((CB, 16), lambda j: (j, 0)),
            pl.BlockSpec((Q, 128), lambda j: (0, 0)),
            pl.BlockSpec((D, H), lambda j: (0, 0)),
            pl.BlockSpec((1, H), lambda j: (0, 0)),
            pl.BlockSpec((H, H), lambda j: (0, 0)),
            pl.BlockSpec((1, H), lambda j: (0, 0)),
            pl.BlockSpec((H, A), lambda j: (0, 0)),
            pl.BlockSpec((1, A), lambda j: (0, 0)),
        ],
        out_specs=pl.BlockSpec((Q, 128), lambda j: (0, 0)),
        out_shape=jax.ShapeDtypeStruct((Q, 128), jnp.int32),
        scratch_shapes=[pltpu.VMEM((Q, 16), jnp.float32)],
    )(dist, observation, valsq, tstar, W1.T, b1[None, :], W2.T, b2[None, :],
      W3.T, b3[None, :])

    return act[:, 0]
